# Initial kernel scaffold; baseline (speedup 1.0000x reference)
import jax
import jax.numpy as jnp
from jax.experimental import pallas as pl


def kernel(atom_xs, atom_edge_index, atom_bipartite_edge_index, atom_edge_xs, atom_coords, subgroup_xs, subgroup_edge_index, subgroup_bipartite_edge_index, subgroup_edge_xs, subgroup_coords, aa_xs, aa_edge_index, aa_bipartite_edge_index, aa_edge_xs, aa_coords, batch, params):
    raise NotImplementedError("write your pallas kernel here")


# speedup vs baseline: 1.5564x; 1.5564x over previous
import functools

import jax
import jax.numpy as jnp
import numpy as np
from jax import lax
from jax.experimental import pallas as pl
from jax.experimental.pallas import tpu as pltpu
from jax.experimental.pallas import tpu_sc as plsc

F32 = jnp.float32
I32 = jnp.int32

NODE = 128
POS = 3
MSGW = 128
EAW = 4
MD = 16
NH = 256
NC = 2
NS = 16
NW = NC * NS
CH = 128


def _sl(x):
    return x * jax.nn.sigmoid(x)


def _ceil_to(a, b):
    return -(-a // b) * b


@functools.lru_cache(maxsize=None)
def _sc_gather_fn(e_pad, n_pad):
    eper = e_pad // NW
    nchunk = eper // CH
    mesh = plsc.VectorSubcoreMesh(core_axis_name="c", subcore_axis_name="s")
    rows = jax.ShapeDtypeStruct((e_pad, NODE), F32)

    @functools.partial(
        pl.kernel,
        out_type=(rows, rows, rows, rows),
        mesh=mesh,
        scratch_types=[
            pltpu.VMEM((CH,), I32),
            pltpu.VMEM((CH,), I32),
            pltpu.VMEM((CH, NODE), F32),
            pltpu.VMEM((CH, NODE), F32),
            pltpu.VMEM((CH, NODE), F32),
            pltpu.VMEM((CH, NODE), F32),
            pltpu.SemaphoreType.DMA,
            pltpu.SemaphoreType.DMA,
            pltpu.SemaphoreType.DMA,
            pltpu.SemaphoreType.DMA,
        ],
    )
    def k(xf_hbm, xc_hbm, src_hbm, dst_hbm,
          gfs_hbm, gfd_hbm, gcs_hbm, gcd_hbm,
          si, di, fs, fd, cs, cd, sem_a, sem_b, sem_c, sem_d):
        cid = lax.axis_index("c")
        sid = lax.axis_index("s")
        base = (sid * NC + cid) * eper

        def body(j, carry):
            off = base + j * CH
            pltpu.sync_copy(src_hbm.at[pl.ds(off, CH)], si)
            pltpu.sync_copy(dst_hbm.at[pl.ds(off, CH)], di)
            a = pltpu.async_copy(xf_hbm.at[si], fs, sem_a)
            b = pltpu.async_copy(xf_hbm.at[di], fd, sem_b)
            c = pltpu.async_copy(xc_hbm.at[si], cs, sem_c)
            d = pltpu.async_copy(xc_hbm.at[di], cd, sem_d)
            a.wait()
            b.wait()
            c.wait()
            d.wait()
            pltpu.sync_copy(fs, gfs_hbm.at[pl.ds(off, CH)])
            pltpu.sync_copy(fd, gfd_hbm.at[pl.ds(off, CH)])
            pltpu.sync_copy(cs, gcs_hbm.at[pl.ds(off, CH)])
            pltpu.sync_copy(cd, gcd_hbm.at[pl.ds(off, CH)])
            return carry

        lax.fori_loop(0, nchunk, body, 0)

    return k


@functools.lru_cache(maxsize=None)
def _sc_scatter_fn(e_pad, n_pad):
    eper = e_pad // NW
    nchunk = eper // CH
    rows_per = n_pad // NS
    nzero = rows_per // 16
    mesh = plsc.VectorSubcoreMesh(core_axis_name="c", subcore_axis_name="s")

    @functools.partial(
        pl.kernel,
        out_type=jax.ShapeDtypeStruct((NC, n_pad, MSGW), F32),
        mesh=mesh,
        scratch_types=[
            pltpu.VMEM((CH,), I32),
            pltpu.VMEM((CH, MSGW), F32),
            pltpu.VMEM((16, MSGW), F32),
            pltpu.VMEM_SHARED((n_pad, MSGW), F32),
        ],
    )
    def k(msg_hbm, dst_hbm, out_hbm, idx_v, msg_v, cbuf, acc_sh):
        cid = lax.axis_index("c")
        sid = lax.axis_index("s")

        z = jnp.zeros((16,), F32)
        for i in range(16):
            for j in range(MSGW // 16):
                cbuf[i, pl.ds(j * 16, 16)] = z

        def zb(t, carry):
            pltpu.sync_copy(cbuf, acc_sh.at[pl.ds(sid * rows_per + t * 16, 16)])
            return carry

        lax.fori_loop(0, nzero, zb, 0)
        plsc.subcore_barrier()

        base = (cid * NS + sid) * eper

        def body(j, carry):
            off = base + j * CH
            pltpu.sync_copy(dst_hbm.at[pl.ds(off, CH)], idx_v)
            pltpu.sync_copy(msg_hbm.at[pl.ds(off, CH)], msg_v)
            pltpu.sync_copy(msg_v, acc_sh.at[idx_v], add=True)
            return carry

        lax.fori_loop(0, nchunk, body, 0)
        plsc.subcore_barrier()

        def cb(t, carry):
            r0 = sid * rows_per + t * 16
            pltpu.sync_copy(acc_sh.at[pl.ds(r0, 16)], cbuf)
            pltpu.sync_copy(cbuf, out_hbm.at[cid].at[pl.ds(r0, 16)])
            return carry

        lax.fori_loop(0, nzero, cb, 0)

    return k


def _tc_edge(gfd, gfs, gcd, gcs, ea, wd, ws, we, wr, b1, w2, b2,
             wc1, bc1, wc2, bc2, emb):
    e_pad = gfd.shape[0]
    be = 512
    grid = (e_pad // be,)

    def body(gfd_r, gfs_r, gcd_r, gcs_r, ea_r, wd_r, ws_r, we_r, wr_r, b1_r,
             w2_r, b2_r, wc1_r, bc1_r, wc2_r, bc2_r, emb_r, out_r):
        rel = gcs_r[:, :POS] - gcd_r[:, :POS]
        rd = jnp.sum(rel * rel, axis=1, keepdims=True)
        h = (jnp.dot(gfd_r[...], wd_r[...], preferred_element_type=F32)
             + jnp.dot(gfs_r[...], ws_r[...], preferred_element_type=F32)
             + jnp.dot(ea_r[...], we_r[...], preferred_element_type=F32)
             + rd * wr_r[...] + b1_r[...])
        h = _sl(h)
        m = _sl(jnp.dot(h, w2_r[...], preferred_element_type=F32) + b2_r[...])
        c = (jnp.dot(_sl(jnp.dot(m, wc1_r[...], preferred_element_type=F32)
                         + bc1_r[...]),
                     wc2_r[...], preferred_element_type=F32) + bc2_r[...])
        out_r[...] = jnp.dot(
            jnp.concatenate([m, c * rel], axis=1), emb_r[...],
            preferred_element_type=F32)

    full = lambda a: pl.BlockSpec(a.shape, lambda i: (0,) * a.ndim)
    args = (gfd, gfs, gcd, gcs, ea, wd, ws, we, wr, b1, w2, b2,
            wc1, bc1, wc2, bc2, emb)
    return pl.pallas_call(
        body,
        grid=grid,
        in_specs=[
            pl.BlockSpec((be, NODE), lambda i: (i, 0)),
            pl.BlockSpec((be, NODE), lambda i: (i, 0)),
            pl.BlockSpec((be, NODE), lambda i: (i, 0)),
            pl.BlockSpec((be, NODE), lambda i: (i, 0)),
            pl.BlockSpec((be, EAW), lambda i: (i, 0)),
        ] + [full(a) for a in args[5:]],
        out_specs=pl.BlockSpec((be, MSGW), lambda i: (i, 0)),
        out_shape=jax.ShapeDtypeStruct((e_pad, MSGW), F32),
    )(*args)


def _tc_node(xf, xc, acc, wf, wm, b1, w2, b2, sel):
    n_pad = xf.shape[0]
    bn = 256
    grid = (n_pad // bn,)

    def body(xf_r, xc_r, acc_r, wf_r, wm_r, b1_r, w2_r, b2_r, sel_r,
             of_r, oc_r):
        a = acc_r[0] + acc_r[1]
        feats = xf_r[...]
        h = _sl(jnp.dot(feats, wf_r[...], preferred_element_type=F32)
                + jnp.dot(a, wm_r[...], preferred_element_type=F32) + b1_r[...])
        h = jnp.dot(h, w2_r[...], preferred_element_type=F32) + b2_r[...]
        of_r[...] = feats + h
        oc_r[...] = xc_r[...] + jnp.dot(a, sel_r[...],
                                        preferred_element_type=F32)

    full = lambda a: pl.BlockSpec(a.shape, lambda i: (0,) * a.ndim)
    out = jax.ShapeDtypeStruct((n_pad, NODE), F32)
    return pl.pallas_call(
        body,
        grid=grid,
        in_specs=[
            pl.BlockSpec((bn, NODE), lambda i: (i, 0)),
            pl.BlockSpec((bn, NODE), lambda i: (i, 0)),
            pl.BlockSpec((NC, bn, MSGW), lambda i: (0, i, 0)),
            full(wf), full(wm), full(b1), full(w2), full(b2), full(sel),
        ],
        out_specs=(pl.BlockSpec((bn, NODE), lambda i: (i, 0)),
                   pl.BlockSpec((bn, NODE), lambda i: (i, 0))),
        out_shape=(out, out),
    )(xf, xc, acc, wf, wm, b1, w2, b2, sel)


_EMB = np.eye(MD + POS, MSGW, dtype=np.float32)
_SEL = (np.eye(MSGW, NODE, k=-MD, dtype=np.float32)
        * (np.arange(MSGW) < MD + POS)[:, None]).astype(np.float32)


def _prep_layer_params(p):
    we1 = p["We1"]
    wn1 = p["Wn1"]
    wm = jnp.concatenate(
        [wn1[NODE:NODE + MD], jnp.zeros((MSGW - MD, NH), F32)], axis=0)
    return dict(
        wd=we1[0:NODE],
        ws=we1[NODE:2 * NODE],
        we=we1[2 * NODE:2 * NODE + EAW],
        wr=we1[2 * NODE + EAW:2 * NODE + EAW + 1],
        b1=p["be1"][None, :],
        w2=p["We2"],
        b2=p["be2"][None, :],
        wc1=p["Wc1"],
        bc1=p["bc1"][None, :],
        wc2=p["Wc2"],
        bc2=p["bc2"][None, :],
        wf=wn1[0:NODE],
        wm=wm,
        nb1=p["bn1"][None, :],
        wn2=p["Wn2"],
        nb2=p["bn2"][None, :],
    )


def _egnn_level(xf, xc, src, dst, ea, plist):
    n_pad = xf.shape[0]
    e_pad = src.shape[0]
    gather = _sc_gather_fn(e_pad, n_pad)
    scatter = _sc_scatter_fn(e_pad, n_pad)
    for p in plist:
        q = _prep_layer_params(p)
        gfs, gfd, gcs, gcd = gather(xf, xc, src, dst)
        msg = _tc_edge(gfd, gfs, gcd, gcs, ea,
                       q["wd"], q["ws"], q["we"], q["wr"], q["b1"],
                       q["w2"], q["b2"], q["wc1"], q["bc1"], q["wc2"],
                       q["bc2"], _EMB)
        acc = scatter(msg, dst)
        xf, xc = _tc_node(xf, xc, acc, q["wf"], q["wm"], q["nb1"],
                          q["wn2"], q["nb2"], _SEL)
    return xf, xc


def _pad_rows(a, n):
    return jnp.concatenate(
        [a, jnp.zeros((n - a.shape[0],) + a.shape[1:], a.dtype)], axis=0)


def _coors128(coors):
    return jnp.concatenate(
        [coors, jnp.zeros((coors.shape[0], NODE - POS), F32)], axis=1)


def _make_edges(src, dst, ea, e_pad, pad_idx):
    e = src.shape[0]
    src = jnp.concatenate([src, jnp.full((e_pad - e,), pad_idx, I32)])
    dst = jnp.concatenate([dst, jnp.full((e_pad - e,), pad_idx, I32)])
    ea = _pad_rows(ea, e_pad)
    return src, dst, ea


def kernel(atom_xs, atom_edge_index, atom_bipartite_edge_index, atom_edge_xs,
           atom_coords, subgroup_xs, subgroup_edge_index,
           subgroup_bipartite_edge_index, subgroup_edge_xs, subgroup_coords,
           aa_xs, aa_edge_index, aa_bipartite_edge_index, aa_edge_xs,
           aa_coords, batch, params):
    n0 = atom_xs.shape[0]
    n1 = subgroup_xs.shape[0]
    n2 = aa_xs.shape[0]

    n_pad0 = _ceil_to(n0, 256)
    e_pad0 = _ceil_to(atom_edge_index.shape[1], NW * CH)
    xf0 = _pad_rows(atom_xs, n_pad0)
    xc0 = _pad_rows(_coors128(atom_coords), n_pad0)
    s0, d0, ea0 = _make_edges(atom_edge_index[0], atom_edge_index[1],
                              atom_edge_xs, e_pad0, n_pad0 - 1)
    xf0, xc0 = _egnn_level(xf0, xc0, s0, d0, ea0, params[0])
    atom_out_feats = xf0[:n0]

    nn1 = n0 + n1
    n_pad1 = _ceil_to(nn1, 256)
    e1 = subgroup_edge_index.shape[1] + subgroup_bipartite_edge_index.shape[1]
    e_pad1 = _ceil_to(e1, NW * CH)
    xf1 = _pad_rows(jnp.concatenate([xf0[:n0], subgroup_xs], axis=0), n_pad1)
    xc1 = _pad_rows(jnp.concatenate(
        [xc0[:n0], _coors128(subgroup_coords)], axis=0), n_pad1)
    s1 = jnp.concatenate([subgroup_edge_index[0] + n0,
                          subgroup_bipartite_edge_index[0]])
    d1 = jnp.concatenate([
        subgroup_edge_index[1] + n0,
        jnp.full((subgroup_bipartite_edge_index.shape[1],), n0, I32)])
    ea1 = jnp.concatenate([
        subgroup_edge_xs,
        jnp.zeros((subgroup_bipartite_edge_index.shape[1], EAW), F32)], axis=0)
    s1, d1, ea1 = _make_edges(s1, d1, ea1, e_pad1, n_pad1 - 1)
    xf1, xc1 = _egnn_level(xf1, xc1, s1, d1, ea1, params[1])
    sx_feats = xf1[:n1]

    nn2 = n1 + n2
    n_pad2 = _ceil_to(nn2, 256)
    e2 = aa_edge_index.shape[1] + aa_bipartite_edge_index.shape[1]
    e_pad2 = _ceil_to(e2, NW * CH)
    xf2 = _pad_rows(jnp.concatenate([xf1[:n1], aa_xs], axis=0), n_pad2)
    xc2 = _pad_rows(jnp.concatenate(
        [xc1[:n1], _coors128(aa_coords)], axis=0), n_pad2)
    s2 = jnp.concatenate([aa_edge_index[0] + n1,
                          aa_bipartite_edge_index[0]])
    d2 = jnp.concatenate([
        aa_edge_index[1] + n1,
        jnp.full((aa_bipartite_edge_index.shape[1],), n1, I32)])
    ea2 = jnp.concatenate([
        aa_edge_xs,
        jnp.zeros((aa_bipartite_edge_index.shape[1], EAW), F32)], axis=0)
    s2, d2, ea2 = _make_edges(s2, d2, ea2, e_pad2, n_pad2 - 1)
    xf2, xc2 = _egnn_level(xf2, xc2, s2, d2, ea2, params[2])
    ax_feats = xf2[:nn2]

    return (atom_out_feats, sx_feats, ax_feats)
